# baseline (speedup 1.0000x reference)
import functools

import jax
import jax.numpy as jnp
from jax import lax
from jax.experimental import pallas as pl
from jax.experimental.pallas import tpu as pltpu
from jax.experimental.pallas import tpu_sc as plsc

_N = 100000
_E = 3200000
_H = 16

_NC = 2
_NS = 16
_NW = _NC * _NS
_B = 256
_VISITS = 6
_CHUNK = 128


def _rsqrt_fast(x):
    i = plsc.bitcast(x, jnp.int32)
    i = jnp.int32(0x5F3759DF) - lax.shift_right_arithmetic(i, 1)
    y = plsc.bitcast(i, jnp.float32)
    for _ in range(3):
        y = y * (1.5 - 0.5 * x * y * y)
    return y


def _acc_stripe_copies(base, size, buf, spmem, hbm, to_hbm):
    full, tail = size // _B, size % _B

    @pl.loop(0, full)
    def _cp(i):
        off = pl.multiple_of(base + i * _B, 8)
        if to_hbm is None:
            pltpu.sync_copy(buf, spmem.at[pl.ds(off, _B), :])
        else:
            pltpu.sync_copy(spmem.at[pl.ds(off, _B), :], buf)
            pltpu.sync_copy(buf, hbm.at[pl.ds(off, _B), :])

    if tail:
        off = pl.multiple_of(base + full * _B, 8)
        if to_hbm is None:
            pltpu.sync_copy(buf.at[pl.ds(0, tail), :],
                            spmem.at[pl.ds(off, tail), :])
        else:
            pltpu.sync_copy(spmem.at[pl.ds(off, tail), :],
                            buf.at[pl.ds(0, tail), :])
            pltpu.sync_copy(buf.at[pl.ds(0, tail), :],
                            hbm.at[pl.ds(off, tail), :])


def _den_stripe_copies(base, size, buf, spmem, hbm, to_hbm):
    full, tail = size // _B, size % _B

    @pl.loop(0, full)
    def _cp(i):
        off = pl.multiple_of(base + i * _B, 8)
        if to_hbm is None:
            pltpu.sync_copy(buf, spmem.at[pl.ds(off, _B)])
        else:
            pltpu.sync_copy(spmem.at[pl.ds(off, _B)], buf)
            pltpu.sync_copy(buf, hbm.at[pl.ds(off, _B)])

    if tail:
        off = pl.multiple_of(base + full * _B, 8)
        if to_hbm is None:
            pltpu.sync_copy(buf.at[pl.ds(0, tail)], spmem.at[pl.ds(off, tail)])
        else:
            pltpu.sync_copy(spmem.at[pl.ds(off, tail)], buf.at[pl.ds(0, tail)])
            pltpu.sync_copy(buf.at[pl.ds(0, tail)], hbm.at[pl.ds(off, tail)])


def _striped(sid, s0, s_last, fn):
    if s_last == s0:
        fn(sid * s0, s0)
    else:
        @pl.when(sid < _NS - 1)
        def _full():
            fn(sid * s0, s0)

        @pl.when(sid == _NS - 1)
        def _last():
            fn((_NS - 1) * s0, s_last)


def _edge_body(shapes, tbl, cidx, beta, acc_out, den_out,
               acc_sh, den_sh, ib, hsb, hdb, evb, bv,
               gsem, ssem, isem):
    n, steps, s0a, sla, s0d, sld = shapes
    cid = lax.axis_index("c")
    sid = lax.axis_index("s")
    iot = lax.iota(jnp.int32, 16)
    nchunks = _B // _CHUNK
    ngrp = _B // 16

    zero16 = jnp.zeros((16,), jnp.float32)
    hs0, ev0 = hsb[0], evb[0]

    @pl.loop(0, hs0.shape[0])
    def _zh(i):
        hs0[i] = zero16

    @pl.loop(0, ev0.shape[0] // 16)
    def _ze(i):
        ev0[pl.ds(i * 16, 16)] = zero16

    pltpu.sync_copy(beta, bv)

    _striped(sid, s0a, sla,
             lambda b, s: _acc_stripe_copies(b, s, hs0, acc_sh, None, None))
    _striped(sid, s0d, sld,
             lambda b, s: _den_stripe_copies(b, s, ev0, den_sh, None, None))

    plsc.subcore_barrier()

    sbase = (cid * _NS + sid) * steps
    beta_vec = bv[...]

    def idx_issue(i, s_abs):
        pltpu.async_copy(cidx.at[sbase + s_abs], ib[i], isem[i])

    def idx_wait(i):
        pltpu.make_async_copy(cidx.at[sbase], ib[i], isem[i]).wait()

    def gather_issue(b, i):
        pltpu.async_copy(tbl.at[ib[i].at[0]], hsb[b], gsem[b])
        pltpu.async_copy(tbl.at[ib[i].at[1]], hdb[b], gsem[b])

    def gather_wait(b, i):
        pltpu.make_async_copy(tbl.at[ib[i].at[0]], hsb[b], gsem[b]).wait()
        pltpu.make_async_copy(tbl.at[ib[i].at[1]], hdb[b], gsem[b]).wait()

    def scatter_issue(b, i):
        pltpu.async_copy(hsb[b], acc_sh.at[ib[i].at[2]], ssem[b], add=True)
        pltpu.async_copy(evb[b], den_sh.at[ib[i].at[2]], ssem[b], add=True)

    def scatter_wait(b, i):
        pltpu.make_async_copy(hsb[b], acc_sh.at[ib[i].at[2]], ssem[b]).wait()
        pltpu.make_async_copy(evb[b], den_sh.at[ib[i].at[2]], ssem[b]).wait()

    def compute(b):
        hs, hd, ev = hsb[b], hdb[b], evb[b]

        @pl.loop(0, ngrp)
        def _grp(g):
            rows = g * 16 + iot
            c = jnp.zeros((16,), jnp.float32)
            ns = jnp.zeros((16,), jnp.float32)
            nd = jnp.zeros((16,), jnp.float32)
            vs_all = []
            for k in range(_H):
                kk = jnp.full((16,), k, jnp.int32)
                vs = plsc.load_gather(hs, [rows, kk])
                vd = plsc.load_gather(hd, [rows, kk])
                vs_all.append(vs)
                c = c + vs * vd
                ns = ns + vs * vs
                nd = nd + vd * vd
            inv_s = _rsqrt_fast(jnp.maximum(ns, 1e-24))
            inv_d = _rsqrt_fast(jnp.maximum(nd, 1e-24))
            e = jnp.exp(beta_vec * (c * inv_s * inv_d))
            ev[pl.ds(g * 16, 16)] = e
            for k in range(_H):
                kk = jnp.full((16,), k, jnp.int32)
                plsc.store_scatter(hs, [rows, kk], e * vs_all[k])

    def visit(s_abs, v, do_pf2, do_pf1, first, last):
        b, b1 = v % 2, (v + 1) % 2
        i0, i1, i2 = v % 3, (v + 1) % 3, (v + 2) % 3
        gather_wait(b, i0)
        compute(b)
        scatter_issue(b, i0)
        if not first:
            scatter_wait(b1, (v - 1) % 3)
        if do_pf2:
            idx_issue(i2, s_abs + 2)
        if do_pf1:
            idx_wait(i1)
            gather_issue(b1, i1)
        if last:
            scatter_wait(b, i0)

    idx_issue(0, 0)
    idx_wait(0)
    gather_issue(0, 0)
    idx_issue(1, 1)
    for v in range(_VISITS):
        visit(v, v, True, True, v == 0, False)

    nrounds = steps // _VISITS

    @pl.loop(1, nrounds - 1)
    def _round(p):
        s0 = p * _VISITS
        for v in range(_VISITS):
            visit(s0 + v, v, True, True, False, False)

    tail0 = (nrounds - 1) * _VISITS
    for v in range(_VISITS):
        visit(tail0 + v, v, v + 2 < _VISITS, v + 1 < _VISITS,
              False, v == _VISITS - 1)

    plsc.subcore_barrier()

    acc_o = acc_out.at[cid]
    den_o = den_out.at[cid]
    _striped(sid, s0a, sla,
             lambda b, s: _acc_stripe_copies(b, s, hs0, acc_sh, acc_o, True))
    _striped(sid, s0d, sld,
             lambda b, s: _den_stripe_copies(b, s, ev0, den_sh, den_o, True))


def _edge_pass(tbl, cidx, beta_vec):
    n = tbl.shape[0]
    e_pad = cidx.shape[0] * _B
    assert cidx.shape[0] % _NW == 0
    steps = cidx.shape[0] // _NW
    assert steps % _VISITS == 0 and steps >= 3 * _VISITS
    s0a = (((n // _NS) + 7) // 8) * 8
    sla = n - (_NS - 1) * s0a
    assert 0 < sla <= s0a and sla % 8 == 0
    s0d = (((n // _NS) + 127) // 128) * 128
    sld = n - (_NS - 1) * s0d
    assert 0 < sld <= s0d and sld % 8 == 0
    shapes = (n, steps, s0a, sla, s0d, sld)

    np_ = n + 8
    vmem = pltpu.VMEM
    f = pl.kernel(
        functools.partial(_edge_body, shapes),
        out_type=(jax.ShapeDtypeStruct((_NC, n, _H), jnp.float32),
                  jax.ShapeDtypeStruct((_NC, n), jnp.float32)),
        mesh=plsc.VectorSubcoreMesh(core_axis_name="c", subcore_axis_name="s"),
        compiler_params=pltpu.CompilerParams(use_tc_tiling_on_sc=False,
                                             needs_layout_passes=False),
        scratch_types=[
            pltpu.VMEM_SHARED((np_, _H), jnp.float32),
            pltpu.VMEM_SHARED((np_,), jnp.float32),
            tuple(vmem((3, _B), jnp.int32) for _ in range(3)),
            tuple(vmem((_B, _H), jnp.float32) for _ in range(2)),
            tuple(vmem((_B, _H), jnp.float32) for _ in range(2)),
            tuple(vmem((_B,), jnp.float32) for _ in range(2)),
            vmem((16,), jnp.float32),
            tuple(pltpu.SemaphoreType.DMA for _ in range(2)),
            tuple(pltpu.SemaphoreType.DMA for _ in range(2)),
            tuple(pltpu.SemaphoreType.DMA for _ in range(3)),
        ],
    )
    return f(tbl, cidx, beta_vec)


def _lin_in_body(x_ref, w_ref, b_ref, o_ref):
    o_ref[...] = jnp.maximum(x_ref[...] * w_ref[...] + b_ref[...], 0.0)


def _self_term(t, beta):
    nsq = jnp.sum(t * t, axis=1, keepdims=True)
    m = jnp.maximum(jnp.sqrt(nsq), 1e-12)
    return jnp.exp(beta * (nsq / (m * m)))


def _fin_body(acc_ref, den_ref, t_ref, beta_ref, o_ref):
    a = acc_ref[0] + acc_ref[1]
    d = den_ref[0] + den_ref[1]
    t = t_ref[...]
    es = _self_term(t, beta_ref[...])
    o_ref[...] = (a + es * t) / (d + es)


def _fin_out_body(acc_ref, den_ref, t_ref, beta_ref, w2_ref, b2_ref, o_ref):
    a = acc_ref[0] + acc_ref[1]
    d = den_ref[0] + den_ref[1]
    t = t_ref[...]
    es = _self_term(t, beta_ref[...])
    t3 = (a + es * t) / (d + es)
    o_ref[...] = jnp.sum(t3 * w2_ref[...], axis=1, keepdims=True) + b2_ref[...]


_BN = 2000


def _lin_in(x, w1r, b1r):
    n = x.shape[0]
    return pl.pallas_call(
        _lin_in_body,
        grid=(n // _BN,),
        in_specs=[pl.BlockSpec((_BN, 1), lambda i: (i, 0)),
                  pl.BlockSpec((1, _H), lambda i: (0, 0)),
                  pl.BlockSpec((1, _H), lambda i: (0, 0))],
        out_specs=pl.BlockSpec((_BN, _H), lambda i: (i, 0)),
        out_shape=jax.ShapeDtypeStruct((n, _H), jnp.float32),
    )(x, w1r, b1r)


def _finalize(acc, den3, t, beta11):
    n = t.shape[0]
    return pl.pallas_call(
        _fin_body,
        grid=(n // _BN,),
        in_specs=[pl.BlockSpec((_NC, _BN, _H), lambda i: (0, i, 0)),
                  pl.BlockSpec((_NC, _BN, 1), lambda i: (0, i, 0)),
                  pl.BlockSpec((_BN, _H), lambda i: (i, 0)),
                  pl.BlockSpec((1, 1), lambda i: (0, 0))],
        out_specs=pl.BlockSpec((_BN, _H), lambda i: (i, 0)),
        out_shape=jax.ShapeDtypeStruct((n, _H), jnp.float32),
    )(acc, den3, t, beta11)


def _finalize_out(acc, den3, t, beta11, w2r, b2r):
    n = t.shape[0]
    return pl.pallas_call(
        _fin_out_body,
        grid=(n // _BN,),
        in_specs=[pl.BlockSpec((_NC, _BN, _H), lambda i: (0, i, 0)),
                  pl.BlockSpec((_NC, _BN, 1), lambda i: (0, i, 0)),
                  pl.BlockSpec((_BN, _H), lambda i: (i, 0)),
                  pl.BlockSpec((1, 1), lambda i: (0, 0)),
                  pl.BlockSpec((1, _H), lambda i: (0, 0)),
                  pl.BlockSpec((1, 1), lambda i: (0, 0))],
        out_specs=pl.BlockSpec((_BN, 1), lambda i: (i, 0)),
        out_shape=jax.ShapeDtypeStruct((n, 1), jnp.float32),
    )(acc, den3, t, beta11, w2r, b2r)


def kernel(x, edge_index, W1, b1, beta1, beta2, W2, b2):
    n, e = _N, _E
    src = edge_index[0].astype(jnp.int32)
    dst = edge_index[1].astype(jnp.int32)
    grain = _NW * _B * _VISITS
    e_pad = ((e + grain - 1) // grain) * grain
    pad = e_pad - e
    ar = jnp.arange(pad, dtype=jnp.int32)
    src_p = jnp.concatenate([src, ar % n])
    dst_p = jnp.concatenate([dst, n + (ar % 8)])
    s3 = src_p.reshape(-1, 1, _B)
    d3 = dst_p.reshape(-1, 1, _B)
    cidx = jnp.concatenate([s3, d3, d3], axis=1)

    b1r = b1.reshape(1, _H)
    w1r = W1.reshape(1, _H)
    beta1v = jnp.broadcast_to(beta1.astype(jnp.float32), (16,))
    beta2v = jnp.broadcast_to(beta2.astype(jnp.float32), (16,))

    h1 = _lin_in(x, w1r, b1r)
    acc1, den1 = _edge_pass(h1, cidx, beta1v)
    t2 = _finalize(acc1, den1.reshape(_NC, n, 1), h1, beta1.reshape(1, 1))
    acc2, den2 = _edge_pass(t2, cidx, beta2v)
    return _finalize_out(acc2, den2.reshape(_NC, n, 1), t2,
                         beta2.reshape(1, 1), W2.reshape(1, _H), b2.reshape(1, 1))


# speedup vs baseline: 1.0488x; 1.0488x over previous
import functools

import jax
import jax.numpy as jnp
from jax import lax
from jax.experimental import pallas as pl
from jax.experimental.pallas import tpu as pltpu
from jax.experimental.pallas import tpu_sc as plsc

_N = 100000
_E = 3200000
_H = 16

_NC = 2
_NS = 16
_NW = _NC * _NS
_B = 256
_VISITS = 6
_CHUNK = 128


def _rsqrt_fast(x):
    i = plsc.bitcast(x, jnp.int32)
    i = jnp.int32(0x5F3759DF) - lax.shift_right_arithmetic(i, 1)
    y = plsc.bitcast(i, jnp.float32)
    for _ in range(2):
        y = y * (1.5 - 0.5 * x * y * y)
    return y


def _acc_stripe_copies(base, size, buf, spmem, hbm, to_hbm):
    full, tail = size // _B, size % _B

    @pl.loop(0, full)
    def _cp(i):
        off = pl.multiple_of(base + i * _B, 8)
        if to_hbm is None:
            pltpu.sync_copy(buf, spmem.at[pl.ds(off, _B), :])
        else:
            pltpu.sync_copy(spmem.at[pl.ds(off, _B), :], buf)
            pltpu.sync_copy(buf, hbm.at[pl.ds(off, _B), :])

    if tail:
        off = pl.multiple_of(base + full * _B, 8)
        if to_hbm is None:
            pltpu.sync_copy(buf.at[pl.ds(0, tail), :],
                            spmem.at[pl.ds(off, tail), :])
        else:
            pltpu.sync_copy(spmem.at[pl.ds(off, tail), :],
                            buf.at[pl.ds(0, tail), :])
            pltpu.sync_copy(buf.at[pl.ds(0, tail), :],
                            hbm.at[pl.ds(off, tail), :])


def _den_stripe_copies(base, size, buf, spmem, hbm, to_hbm):
    full, tail = size // _B, size % _B

    @pl.loop(0, full)
    def _cp(i):
        off = pl.multiple_of(base + i * _B, 8)
        if to_hbm is None:
            pltpu.sync_copy(buf, spmem.at[pl.ds(off, _B)])
        else:
            pltpu.sync_copy(spmem.at[pl.ds(off, _B)], buf)
            pltpu.sync_copy(buf, hbm.at[pl.ds(off, _B)])

    if tail:
        off = pl.multiple_of(base + full * _B, 8)
        if to_hbm is None:
            pltpu.sync_copy(buf.at[pl.ds(0, tail)], spmem.at[pl.ds(off, tail)])
        else:
            pltpu.sync_copy(spmem.at[pl.ds(off, tail)], buf.at[pl.ds(0, tail)])
            pltpu.sync_copy(buf.at[pl.ds(0, tail)], hbm.at[pl.ds(off, tail)])


def _striped(sid, s0, s_last, fn):
    if s_last == s0:
        fn(sid * s0, s0)
    else:
        @pl.when(sid < _NS - 1)
        def _full():
            fn(sid * s0, s0)

        @pl.when(sid == _NS - 1)
        def _last():
            fn((_NS - 1) * s0, s_last)


def _edge_body(shapes, tbl, cidx, beta, acc_out, den_out,
               acc_sh, den_sh, ib, hsb, hdb, evb, bv,
               gsem, ssem, isem):
    n, steps, s0a, sla, s0d, sld = shapes
    cid = lax.axis_index("c")
    sid = lax.axis_index("s")
    iot = lax.iota(jnp.int32, 16)
    nchunks = _B // _CHUNK
    ngrp = _B // 16

    zero16 = jnp.zeros((16,), jnp.float32)
    hs0, ev0 = hsb[0], evb[0]

    @pl.loop(0, hs0.shape[0])
    def _zh(i):
        hs0[i] = zero16

    @pl.loop(0, ev0.shape[0] // 16)
    def _ze(i):
        ev0[pl.ds(i * 16, 16)] = zero16

    pltpu.sync_copy(beta, bv)

    _striped(sid, s0a, sla,
             lambda b, s: _acc_stripe_copies(b, s, hs0, acc_sh, None, None))
    _striped(sid, s0d, sld,
             lambda b, s: _den_stripe_copies(b, s, ev0, den_sh, None, None))

    plsc.subcore_barrier()

    sbase = (cid * _NS + sid) * steps
    beta_vec = bv[...]

    def idx_issue(i, s_abs):
        pltpu.async_copy(cidx.at[sbase + s_abs], ib[i], isem[i])

    def idx_wait(i):
        pltpu.make_async_copy(cidx.at[sbase], ib[i], isem[i]).wait()

    def gather_issue(b, i):
        pltpu.async_copy(tbl.at[ib[i].at[0]], hsb[b], gsem[b])
        pltpu.async_copy(tbl.at[ib[i].at[1]], hdb[b], gsem[b])

    def gather_wait(b, i):
        pltpu.make_async_copy(tbl.at[ib[i].at[0]], hsb[b], gsem[b]).wait()
        pltpu.make_async_copy(tbl.at[ib[i].at[1]], hdb[b], gsem[b]).wait()

    def scatter_issue(b, i):
        pltpu.async_copy(hsb[b], acc_sh.at[ib[i].at[2]], ssem[b], add=True)
        pltpu.async_copy(evb[b], den_sh.at[ib[i].at[2]], ssem[b], add=True)

    def scatter_wait(b, i):
        pltpu.make_async_copy(hsb[b], acc_sh.at[ib[i].at[2]], ssem[b]).wait()
        pltpu.make_async_copy(evb[b], den_sh.at[ib[i].at[2]], ssem[b]).wait()

    def compute(b):
        hs, hd, ev = hsb[b], hdb[b], evb[b]

        @pl.loop(0, ngrp, unroll=2)
        def _grp(g):
            rows = g * 16 + iot
            cp = [jnp.zeros((16,), jnp.float32) for _ in range(4)]
            nsp = [jnp.zeros((16,), jnp.float32) for _ in range(4)]
            ndp = [jnp.zeros((16,), jnp.float32) for _ in range(4)]
            vs_all = []
            for k in range(_H):
                kk = jnp.full((16,), k, jnp.int32)
                vs = plsc.load_gather(hs, [rows, kk])
                vd = plsc.load_gather(hd, [rows, kk])
                vs_all.append(vs)
                p = k % 4
                cp[p] = cp[p] + vs * vd
                nsp[p] = nsp[p] + vs * vs
                ndp[p] = ndp[p] + vd * vd
            c = (cp[0] + cp[1]) + (cp[2] + cp[3])
            ns = (nsp[0] + nsp[1]) + (nsp[2] + nsp[3])
            nd = (ndp[0] + ndp[1]) + (ndp[2] + ndp[3])
            inv_s = _rsqrt_fast(jnp.maximum(ns, 1e-24))
            inv_d = _rsqrt_fast(jnp.maximum(nd, 1e-24))
            e = jnp.exp(beta_vec * (c * inv_s * inv_d))
            ev[pl.ds(g * 16, 16)] = e
            for k in range(_H):
                kk = jnp.full((16,), k, jnp.int32)
                plsc.store_scatter(hs, [rows, kk], e * vs_all[k])

    def visit(s_abs, v, do_pf2, do_pf1, first, last):
        b, b1 = v % 2, (v + 1) % 2
        i0, i1, i2 = v % 3, (v + 1) % 3, (v + 2) % 3
        gather_wait(b, i0)
        compute(b)
        scatter_issue(b, i0)
        if not first:
            scatter_wait(b1, (v - 1) % 3)
        if do_pf2:
            idx_issue(i2, s_abs + 2)
        if do_pf1:
            idx_wait(i1)
            gather_issue(b1, i1)
        if last:
            scatter_wait(b, i0)

    idx_issue(0, 0)
    idx_wait(0)
    gather_issue(0, 0)
    idx_issue(1, 1)
    for v in range(_VISITS):
        visit(v, v, True, True, v == 0, False)

    nrounds = steps // _VISITS

    @pl.loop(1, nrounds - 1)
    def _round(p):
        s0 = p * _VISITS
        for v in range(_VISITS):
            visit(s0 + v, v, True, True, False, False)

    tail0 = (nrounds - 1) * _VISITS
    for v in range(_VISITS):
        visit(tail0 + v, v, v + 2 < _VISITS, v + 1 < _VISITS,
              False, v == _VISITS - 1)

    plsc.subcore_barrier()

    acc_o = acc_out.at[cid]
    den_o = den_out.at[cid]
    _striped(sid, s0a, sla,
             lambda b, s: _acc_stripe_copies(b, s, hs0, acc_sh, acc_o, True))
    _striped(sid, s0d, sld,
             lambda b, s: _den_stripe_copies(b, s, ev0, den_sh, den_o, True))


def _edge_pass(tbl, cidx, beta_vec):
    n = tbl.shape[0]
    e_pad = cidx.shape[0] * _B
    assert cidx.shape[0] % _NW == 0
    steps = cidx.shape[0] // _NW
    assert steps % _VISITS == 0 and steps >= 3 * _VISITS
    s0a = (((n // _NS) + 7) // 8) * 8
    sla = n - (_NS - 1) * s0a
    assert 0 < sla <= s0a and sla % 8 == 0
    s0d = (((n // _NS) + 127) // 128) * 128
    sld = n - (_NS - 1) * s0d
    assert 0 < sld <= s0d and sld % 8 == 0
    shapes = (n, steps, s0a, sla, s0d, sld)

    np_ = n + 8
    vmem = pltpu.VMEM
    f = pl.kernel(
        functools.partial(_edge_body, shapes),
        out_type=(jax.ShapeDtypeStruct((_NC, n, _H), jnp.float32),
                  jax.ShapeDtypeStruct((_NC, n), jnp.float32)),
        mesh=plsc.VectorSubcoreMesh(core_axis_name="c", subcore_axis_name="s"),
        compiler_params=pltpu.CompilerParams(use_tc_tiling_on_sc=False,
                                             needs_layout_passes=False),
        scratch_types=[
            pltpu.VMEM_SHARED((np_, _H), jnp.float32),
            pltpu.VMEM_SHARED((np_,), jnp.float32),
            tuple(vmem((3, _B), jnp.int32) for _ in range(3)),
            tuple(vmem((_B, _H), jnp.float32) for _ in range(2)),
            tuple(vmem((_B, _H), jnp.float32) for _ in range(2)),
            tuple(vmem((_B,), jnp.float32) for _ in range(2)),
            vmem((16,), jnp.float32),
            tuple(pltpu.SemaphoreType.DMA for _ in range(2)),
            tuple(pltpu.SemaphoreType.DMA for _ in range(2)),
            tuple(pltpu.SemaphoreType.DMA for _ in range(3)),
        ],
    )
    return f(tbl, cidx, beta_vec)


def _lin_in_body(x_ref, w_ref, b_ref, o_ref):
    o_ref[...] = jnp.maximum(x_ref[...] * w_ref[...] + b_ref[...], 0.0)


def _self_term(t, beta):
    nsq = jnp.sum(t * t, axis=1, keepdims=True)
    m = jnp.maximum(jnp.sqrt(nsq), 1e-12)
    return jnp.exp(beta * (nsq / (m * m)))


def _fin_body(acc_ref, den_ref, t_ref, beta_ref, o_ref):
    a = acc_ref[0] + acc_ref[1]
    d = den_ref[0] + den_ref[1]
    t = t_ref[...]
    es = _self_term(t, beta_ref[...])
    o_ref[...] = (a + es * t) / (d + es)


def _fin_out_body(acc_ref, den_ref, t_ref, beta_ref, w2_ref, b2_ref, o_ref):
    a = acc_ref[0] + acc_ref[1]
    d = den_ref[0] + den_ref[1]
    t = t_ref[...]
    es = _self_term(t, beta_ref[...])
    t3 = (a + es * t) / (d + es)
    o_ref[...] = jnp.sum(t3 * w2_ref[...], axis=1, keepdims=True) + b2_ref[...]


_BN = 2000


def _lin_in(x, w1r, b1r):
    n = x.shape[0]
    return pl.pallas_call(
        _lin_in_body,
        grid=(n // _BN,),
        in_specs=[pl.BlockSpec((_BN, 1), lambda i: (i, 0)),
                  pl.BlockSpec((1, _H), lambda i: (0, 0)),
                  pl.BlockSpec((1, _H), lambda i: (0, 0))],
        out_specs=pl.BlockSpec((_BN, _H), lambda i: (i, 0)),
        out_shape=jax.ShapeDtypeStruct((n, _H), jnp.float32),
    )(x, w1r, b1r)


def _finalize(acc, den3, t, beta11):
    n = t.shape[0]
    return pl.pallas_call(
        _fin_body,
        grid=(n // _BN,),
        in_specs=[pl.BlockSpec((_NC, _BN, _H), lambda i: (0, i, 0)),
                  pl.BlockSpec((_NC, _BN, 1), lambda i: (0, i, 0)),
                  pl.BlockSpec((_BN, _H), lambda i: (i, 0)),
                  pl.BlockSpec((1, 1), lambda i: (0, 0))],
        out_specs=pl.BlockSpec((_BN, _H), lambda i: (i, 0)),
        out_shape=jax.ShapeDtypeStruct((n, _H), jnp.float32),
    )(acc, den3, t, beta11)


def _finalize_out(acc, den3, t, beta11, w2r, b2r):
    n = t.shape[0]
    return pl.pallas_call(
        _fin_out_body,
        grid=(n // _BN,),
        in_specs=[pl.BlockSpec((_NC, _BN, _H), lambda i: (0, i, 0)),
                  pl.BlockSpec((_NC, _BN, 1), lambda i: (0, i, 0)),
                  pl.BlockSpec((_BN, _H), lambda i: (i, 0)),
                  pl.BlockSpec((1, 1), lambda i: (0, 0)),
                  pl.BlockSpec((1, _H), lambda i: (0, 0)),
                  pl.BlockSpec((1, 1), lambda i: (0, 0))],
        out_specs=pl.BlockSpec((_BN, 1), lambda i: (i, 0)),
        out_shape=jax.ShapeDtypeStruct((n, 1), jnp.float32),
    )(acc, den3, t, beta11, w2r, b2r)


def kernel(x, edge_index, W1, b1, beta1, beta2, W2, b2):
    n, e = _N, _E
    src = edge_index[0].astype(jnp.int32)
    dst = edge_index[1].astype(jnp.int32)
    grain = _NW * _B * _VISITS
    e_pad = ((e + grain - 1) // grain) * grain
    pad = e_pad - e
    ar = jnp.arange(pad, dtype=jnp.int32)
    src_p = jnp.concatenate([src, ar % n])
    dst_p = jnp.concatenate([dst, n + (ar % 8)])
    s3 = src_p.reshape(-1, 1, _B)
    d3 = dst_p.reshape(-1, 1, _B)
    cidx = jnp.concatenate([s3, d3, d3], axis=1)

    b1r = b1.reshape(1, _H)
    w1r = W1.reshape(1, _H)
    beta1v = jnp.broadcast_to(beta1.astype(jnp.float32), (16,))
    beta2v = jnp.broadcast_to(beta2.astype(jnp.float32), (16,))

    h1 = _lin_in(x, w1r, b1r)
    acc1, den1 = _edge_pass(h1, cidx, beta1v)
    t2 = _finalize(acc1, den1.reshape(_NC, n, 1), h1, beta1.reshape(1, 1))
    acc2, den2 = _edge_pass(t2, cidx, beta2v)
    return _finalize_out(acc2, den2.reshape(_NC, n, 1), t2,
                         beta2.reshape(1, 1), W2.reshape(1, _H), b2.reshape(1, 1))
